# baseline (speedup 1.0000x reference)
import functools

import jax
import jax.numpy as jnp
from jax import lax
from jax.experimental import pallas as pl
from jax.experimental.pallas import tpu as pltpu
from jax.experimental.pallas import tpu_sc as plsc


_GELU_Q = (-4.5212428e-07, 2.6004855e-05, -6.155766e-04, 7.9248846e-03,
           -6.311591e-02, 3.9731035e-01)
_GELU_XC = 4.0


def _gelu(x):
    xc = jnp.clip(x, -_GELU_XC, _GELU_XC)
    s = xc * xc
    acc = jnp.full_like(s, _GELU_Q[0])
    for c in _GELU_Q[1:]:
        acc = acc * s + c
    return x * (acc * xc + 0.5)


def _stage1_body(h_ref, p_ref, w1_ref, b1_ref, a_ref, g_ref):
    C = h_ref.shape[1]
    hb = h_ref[...]
    w1a = w1_ref[0:C, :]
    w1b = w1_ref[C:2 * C, :]
    r = (p_ref[:, 0:1] * w1_ref[2 * C:2 * C + 1, :]
         + p_ref[:, 1:2] * w1_ref[2 * C + 1:2 * C + 2, :]
         + p_ref[:, 2:3] * w1_ref[2 * C + 2:2 * C + 3, :])
    g_ref[...] = jnp.dot(hb, w1b, preferred_element_type=jnp.float32) + r
    a_ref[...] = (jnp.dot(hb, w1a - w1b, preferred_element_type=jnp.float32)
                  + b1_ref[...] - r)


def _stage3_body(nodes, k, h_out, gath_ref, a_ref, w2_ref, b2_ref,
                 gm_ref, bt_ref, o_ref):
    z = gath_ref[...] + a_ref[...][None]
    x1 = _gelu(z).reshape(k * nodes, h_out)
    y = jnp.dot(x1, w2_ref[...], preferred_element_type=jnp.float32)
    y3 = y.reshape(k, nodes, h_out)
    ymax = jnp.max(y3, axis=0) + b2_ref[...]
    ymin = jnp.min(y3, axis=0) + b2_ref[...]
    m = jnp.maximum(_gelu(ymax), _gelu(ymin))
    mu = jnp.mean(m, axis=-1, keepdims=True)
    var = jnp.mean((m - mu) ** 2, axis=-1, keepdims=True)
    o_ref[...] = (m - mu) * lax.rsqrt(var + 1e-5) * gm_ref[...] + bt_ref[...]


def _make_sc_gather(n_edges, H, NC, NS):
    NW = NC * NS
    per_w = n_edges // NW
    CHUNK = 80
    NBUF = 5
    n_groups = per_w // (CHUNK * NBUF)
    tail = []
    o = n_groups * CHUNK * NBUF
    while o < per_w:
        c = min(CHUNK, per_w - o)
        assert c % 8 == 0
        tail.append((o, c))
        o += c
    assert len(tail) <= NBUF

    mesh = plsc.VectorSubcoreMesh(core_axis_name="c", subcore_axis_name="s")

    @functools.partial(
        pl.kernel,
        out_type=jax.ShapeDtypeStruct((n_edges, H), jnp.float32),
        mesh=mesh,
        scratch_types=[
            pltpu.VMEM((per_w,), jnp.int32),
            pltpu.VMEM((2, NBUF, CHUNK, H), jnp.float32),
            pltpu.SemaphoreType.DMA,
            pltpu.SemaphoreType.DMA,
        ],
    )
    def _gather(idx_hbm, g_hbm, out_hbm, idx_v, rows_v, sem_g, sem_s):
        wid = lax.axis_index("s") * NC + lax.axis_index("c")
        base0 = wid * per_w
        pltpu.sync_copy(idx_hbm.at[pl.ds(base0, per_w)], idx_v)

        def drain_stores(par):
            for b in range(NBUF):
                pltpu.make_async_copy(
                    rows_v.at[par, b],
                    out_hbm.at[pl.ds(base0, CHUNK)], sem_s).wait()

        def body(t, carry):
            par = lax.rem(t, 2)
            @pl.when(t >= 2)
            def _():
                drain_stores(par)
            gets = []
            for b in range(NBUF):
                off = (t * NBUF + b) * CHUNK
                gets.append(pltpu.async_copy(
                    g_hbm.at[idx_v.at[pl.ds(off, CHUNK)]],
                    rows_v.at[par, b], sem_g))
            for c in gets:
                c.wait()
            for b in range(NBUF):
                off = (t * NBUF + b) * CHUNK
                pltpu.async_copy(rows_v.at[par, b],
                                 out_hbm.at[pl.ds(base0 + off, CHUNK)], sem_s)
            return carry

        lax.fori_loop(0, n_groups, body, 0)
        drain_stores(lax.rem(jnp.int32(n_groups), 2))
        if n_groups >= 2:
            drain_stores(lax.rem(jnp.int32(n_groups) + 1, 2))
        if tail:
            gets = []
            for j, (off, cnt) in enumerate(tail):
                gets.append(pltpu.async_copy(
                    g_hbm.at[idx_v.at[pl.ds(off, cnt)]],
                    rows_v.at[0, j, pl.ds(0, cnt)], sem_g))
            for c in gets:
                c.wait()
            puts = []
            for j, (off, cnt) in enumerate(tail):
                puts.append(pltpu.async_copy(
                    rows_v.at[0, j, pl.ds(0, cnt)],
                    out_hbm.at[pl.ds(base0 + off, cnt)], sem_s))
            for c in puts:
                c.wait()

    return _gather


def kernel(h, pos, idx, W1, b1, W2, b2, gamma, beta):
    B, N, C = h.shape
    K = idx.shape[-1]
    H = W1.shape[1]
    OUT = W2.shape[1]
    BN = B * N

    hf = h.reshape(BN, C)
    pf = pos.reshape(BN, 3)

    BLK1 = 1000
    grid1 = BN // BLK1
    a, g = pl.pallas_call(
        _stage1_body,
        grid=(grid1,),
        in_specs=[
            pl.BlockSpec((BLK1, C), lambda i: (i, 0)),
            pl.BlockSpec((BLK1, 3), lambda i: (i, 0)),
            pl.BlockSpec((2 * C + 3, H), lambda i: (0, 0)),
            pl.BlockSpec((1, H), lambda i: (0, 0)),
        ],
        out_specs=[
            pl.BlockSpec((BLK1, H), lambda i: (i, 0)),
            pl.BlockSpec((BLK1, H), lambda i: (i, 0)),
        ],
        out_shape=[
            jax.ShapeDtypeStruct((BN, H), jnp.float32),
            jax.ShapeDtypeStruct((BN, H), jnp.float32),
        ],
    )(hf, pf, W1, b1.reshape(1, H))

    info = plsc.get_sparse_core_info()
    NC, NS = info.num_cores, info.num_subcores
    E_b = N * K
    sc_gather = _make_sc_gather(E_b, H, NC, NS)

    gathered = []
    for b in range(B):
        idxT_b = idx[b].T.reshape(E_b)
        g_b = lax.slice_in_dim(g, b * N, (b + 1) * N, axis=0)
        gathered.append(sc_gather(idxT_b, g_b))

    BLKN = 200
    grid3 = N // BLKN
    b2r = b2.reshape(1, OUT)
    gmr = gamma.reshape(1, OUT)
    btr = beta.reshape(1, OUT)

    outs = []
    for b in range(B):
        nblk = N // BLKN
        out_b = pl.pallas_call(
            functools.partial(_stage3_body, BLKN, K, H),
            grid=(grid3,),
            in_specs=[
                pl.BlockSpec((K, BLKN, H), lambda i: (0, i, 0)),
                pl.BlockSpec((BLKN, H), lambda i, o=b * nblk: (i + o, 0)),
                pl.BlockSpec((H, OUT), lambda i: (0, 0)),
                pl.BlockSpec((1, OUT), lambda i: (0, 0)),
                pl.BlockSpec((1, OUT), lambda i: (0, 0)),
                pl.BlockSpec((1, OUT), lambda i: (0, 0)),
            ],
            out_specs=pl.BlockSpec((BLKN, OUT), lambda i: (i, 0)),
            out_shape=jax.ShapeDtypeStruct((N, OUT), jnp.float32),
        )(gathered[b].reshape(K, N, H), a, W2, b2r, gmr, btr)
        outs.append(out_b)

    return jnp.stack(outs)


# speedup vs baseline: 1.0201x; 1.0201x over previous
import functools

import jax
import jax.numpy as jnp
from jax import lax
from jax.experimental import pallas as pl
from jax.experimental.pallas import tpu as pltpu
from jax.experimental.pallas import tpu_sc as plsc


_GELU_Q = (-4.5212428e-07, 2.6004855e-05, -6.155766e-04, 7.9248846e-03,
           -6.311591e-02, 3.9731035e-01)
_GELU_XC = 4.0


def _gelu(x):
    xc = jnp.clip(x, -_GELU_XC, _GELU_XC)
    s = xc * xc
    acc = jnp.full_like(s, _GELU_Q[0])
    for c in _GELU_Q[1:]:
        acc = acc * s + c
    return x * (acc * xc + 0.5)


def _stage1_body(h_ref, p_ref, w1_ref, b1_ref, a_ref, g_ref):
    C = h_ref.shape[1]
    hb = h_ref[...]
    w1a = w1_ref[0:C, :]
    w1b = w1_ref[C:2 * C, :]
    r = (p_ref[:, 0:1] * w1_ref[2 * C:2 * C + 1, :]
         + p_ref[:, 1:2] * w1_ref[2 * C + 1:2 * C + 2, :]
         + p_ref[:, 2:3] * w1_ref[2 * C + 2:2 * C + 3, :])
    g_ref[...] = jnp.dot(hb, w1b, preferred_element_type=jnp.float32) + r
    a_ref[...] = (jnp.dot(hb, w1a - w1b, preferred_element_type=jnp.float32)
                  + b1_ref[...] - r)


def _stage3_body(nodes, k, h_out, gath_ref, a_ref, w2_ref, b2_ref,
                 gm_ref, bt_ref, o_ref):
    z = gath_ref[...] + a_ref[...][None]
    x1 = _gelu(z).reshape(k * nodes, h_out)
    y = jnp.dot(x1, w2_ref[...], preferred_element_type=jnp.float32)
    y3 = y.reshape(k, nodes, h_out)
    ymax = jnp.max(y3, axis=0) + b2_ref[...]
    ymin = jnp.min(y3, axis=0) + b2_ref[...]
    m = jnp.maximum(_gelu(ymax), _gelu(ymin))
    mu = jnp.mean(m, axis=-1, keepdims=True)
    var = jnp.mean((m - mu) ** 2, axis=-1, keepdims=True)
    o_ref[...] = (m - mu) * lax.rsqrt(var + 1e-5) * gm_ref[...] + bt_ref[...]


def _make_sc_gather(n_edges, H, NC, NS, row_off, n_rows):
    NW = NC * NS
    per_w = n_edges // NW
    CHUNK = 80
    NBUF = 5
    n_groups = per_w // (CHUNK * NBUF)
    tail = []
    o = n_groups * CHUNK * NBUF
    while o < per_w:
        c = min(CHUNK, per_w - o)
        assert c % 8 == 0
        tail.append((o, c))
        o += c
    assert len(tail) <= NBUF

    mesh = plsc.VectorSubcoreMesh(core_axis_name="c", subcore_axis_name="s")

    @functools.partial(
        pl.kernel,
        out_type=jax.ShapeDtypeStruct((n_edges, H), jnp.float32),
        mesh=mesh,
        scratch_types=[
            pltpu.VMEM((per_w,), jnp.int32),
            pltpu.VMEM((2, NBUF, CHUNK, H), jnp.float32),
            pltpu.SemaphoreType.DMA,
            pltpu.SemaphoreType.DMA,
        ],
    )
    def _gather(idx_hbm, g_full_hbm, out_hbm, idx_v, rows_v, sem_g, sem_s):
        wid = lax.axis_index("s") * NC + lax.axis_index("c")
        base0 = wid * per_w
        g_hbm = g_full_hbm.at[pl.ds(row_off, n_rows)]
        pltpu.sync_copy(idx_hbm.at[pl.ds(base0, per_w)], idx_v)

        def drain_stores(par):
            for b in range(NBUF):
                pltpu.make_async_copy(
                    rows_v.at[par, b],
                    out_hbm.at[pl.ds(base0, CHUNK)], sem_s).wait()

        def body(t, carry):
            par = lax.rem(t, 2)
            @pl.when(t >= 2)
            def _():
                drain_stores(par)
            gets = []
            for b in range(NBUF):
                off = (t * NBUF + b) * CHUNK
                gets.append(pltpu.async_copy(
                    g_hbm.at[idx_v.at[pl.ds(off, CHUNK)]],
                    rows_v.at[par, b], sem_g))
            for c in gets:
                c.wait()
            for b in range(NBUF):
                off = (t * NBUF + b) * CHUNK
                pltpu.async_copy(rows_v.at[par, b],
                                 out_hbm.at[pl.ds(base0 + off, CHUNK)], sem_s)
            return carry

        lax.fori_loop(0, n_groups, body, 0)
        drain_stores(lax.rem(jnp.int32(n_groups), 2))
        if n_groups >= 2:
            drain_stores(lax.rem(jnp.int32(n_groups) + 1, 2))
        if tail:
            gets = []
            for j, (off, cnt) in enumerate(tail):
                gets.append(pltpu.async_copy(
                    g_hbm.at[idx_v.at[pl.ds(off, cnt)]],
                    rows_v.at[0, j, pl.ds(0, cnt)], sem_g))
            for c in gets:
                c.wait()
            puts = []
            for j, (off, cnt) in enumerate(tail):
                puts.append(pltpu.async_copy(
                    rows_v.at[0, j, pl.ds(0, cnt)],
                    out_hbm.at[pl.ds(base0 + off, cnt)], sem_s))
            for c in puts:
                c.wait()

    return _gather


def kernel(h, pos, idx, W1, b1, W2, b2, gamma, beta):
    B, N, C = h.shape
    K = idx.shape[-1]
    H = W1.shape[1]
    OUT = W2.shape[1]
    BN = B * N

    hf = h.reshape(BN, C)
    pf = pos.reshape(BN, 3)

    BLK1 = 1000
    grid1 = BN // BLK1
    a, g = pl.pallas_call(
        _stage1_body,
        grid=(grid1,),
        in_specs=[
            pl.BlockSpec((BLK1, C), lambda i: (i, 0)),
            pl.BlockSpec((BLK1, 3), lambda i: (i, 0)),
            pl.BlockSpec((2 * C + 3, H), lambda i: (0, 0)),
            pl.BlockSpec((1, H), lambda i: (0, 0)),
        ],
        out_specs=[
            pl.BlockSpec((BLK1, H), lambda i: (i, 0)),
            pl.BlockSpec((BLK1, H), lambda i: (i, 0)),
        ],
        out_shape=[
            jax.ShapeDtypeStruct((BN, H), jnp.float32),
            jax.ShapeDtypeStruct((BN, H), jnp.float32),
        ],
    )(hf, pf, W1, b1.reshape(1, H))

    info = plsc.get_sparse_core_info()
    NC, NS = info.num_cores, info.num_subcores
    E_b = N * K
    gathered = []
    for b in range(B):
        idxT_b = idx[b].T.reshape(E_b)
        sc_gather = _make_sc_gather(E_b, H, NC, NS, b * N, N)
        gathered.append(sc_gather(idxT_b, g))

    BLKN = 200
    grid3 = N // BLKN
    b2r = b2.reshape(1, OUT)
    gmr = gamma.reshape(1, OUT)
    btr = beta.reshape(1, OUT)

    outs = []
    for b in range(B):
        nblk = N // BLKN
        out_b = pl.pallas_call(
            functools.partial(_stage3_body, BLKN, K, H),
            grid=(grid3,),
            in_specs=[
                pl.BlockSpec((K, BLKN, H), lambda i: (0, i, 0)),
                pl.BlockSpec((BLKN, H), lambda i, o=b * nblk: (i + o, 0)),
                pl.BlockSpec((H, OUT), lambda i: (0, 0)),
                pl.BlockSpec((1, OUT), lambda i: (0, 0)),
                pl.BlockSpec((1, OUT), lambda i: (0, 0)),
                pl.BlockSpec((1, OUT), lambda i: (0, 0)),
            ],
            out_specs=pl.BlockSpec((BLKN, OUT), lambda i: (i, 0)),
            out_shape=jax.ShapeDtypeStruct((N, OUT), jnp.float32),
        )(gathered[b].reshape(K, N, H), a, W2, b2r, gmr, btr)
        outs.append(out_b)

    return jnp.stack(outs)
